# Initial kernel scaffold; baseline (speedup 1.0000x reference)
import jax
import jax.numpy as jnp
from jax.experimental import pallas as pl


def kernel(x, token_table, position_table):
    raise NotImplementedError("write your pallas kernel here")


# speedup vs baseline: 1.1911x; 1.1911x over previous
import functools

import jax
import jax.numpy as jnp
from jax import lax
from jax.experimental import pallas as pl
from jax.experimental.pallas import tpu as pltpu
from jax.experimental.pallas import tpu_sc as plsc

BATCH = 4
SEQ_LEN = 2048
D_MODEL = 768

_NUM_CORES = 2
_NUM_SUBCORES = 16
_NW = _NUM_CORES * _NUM_SUBCORES
_S_PER_W = SEQ_LEN // _NW
_LANES = 16
_D_SLICES = D_MODEL // _LANES


def _body(x_hbm, tok_hbm, pos_hbm, out_hbm, idx_v, tok_v, pos_v, sem):
    wid = lax.axis_index("s") * _NUM_CORES + lax.axis_index("c")
    s_base = wid * _S_PER_W

    pltpu.sync_copy(pos_hbm.at[pl.ds(s_base, _S_PER_W)], pos_v)
    for b in range(BATCH):
        pltpu.sync_copy(x_hbm.at[b, pl.ds(s_base, _S_PER_W)], idx_v.at[b])

    def per_batch(b, _):
        pltpu.async_copy(tok_hbm.at[idx_v.at[b]], tok_v, sem).wait()

        def per_row(r, _):
            for j in range(_D_SLICES):
                sl = pl.ds(j * _LANES, _LANES)
                tok_v[r, sl] = tok_v[r, sl] + pos_v[r, sl]
            return 0

        lax.fori_loop(0, _S_PER_W, per_row, 0, unroll=False)
        pltpu.sync_copy(tok_v, out_hbm.at[b, pl.ds(s_base, _S_PER_W)])
        return 0

    lax.fori_loop(0, BATCH, per_batch, 0, unroll=False)


@functools.partial(
    pl.kernel,
    out_type=jax.ShapeDtypeStruct((BATCH, SEQ_LEN, D_MODEL), jnp.float32),
    mesh=plsc.VectorSubcoreMesh(core_axis_name="c", subcore_axis_name="s"),
    scratch_types=[
        pltpu.VMEM((BATCH, _S_PER_W), jnp.int32),
        pltpu.VMEM((_S_PER_W, D_MODEL), jnp.float32),
        pltpu.VMEM((_S_PER_W, D_MODEL), jnp.float32),
        pltpu.SemaphoreType.DMA,
    ],
)
def _emb_lookup(x_hbm, tok_hbm, pos_hbm, out_hbm, idx_v, tok_v, pos_v, sem):
    _body(x_hbm, tok_hbm, pos_hbm, out_hbm, idx_v, tok_v, pos_v, sem)


def kernel(x, token_table, position_table):
    x = x.astype(jnp.int32)
    return _emb_lookup(x, token_table, position_table)
